# Initial kernel scaffold; baseline (speedup 1.0000x reference)
import jax
import jax.numpy as jnp
from jax.experimental import pallas as pl


def kernel(pred, rank_batch):
    raise NotImplementedError("write your pallas kernel here")


# speedup vs baseline: 8035.4538x; 8035.4538x over previous
import functools

import jax
import jax.numpy as jnp
from jax import lax
from jax.experimental import pallas as pl
from jax.experimental.pallas import tpu as pltpu
from jax.experimental.pallas import tpu_sc as plsc

B = 4096
L = 16
NC = 2
NS = 16
NW = NC * NS
ROWS = B // NW
CHUNKS = B // L

_mesh = plsc.VectorSubcoreMesh(core_axis_name="c", subcore_axis_name="s")


@functools.partial(
    pl.kernel,
    mesh=_mesh,
    out_type=[
        jax.ShapeDtypeStruct((NW, L), jnp.float32),
        jax.ShapeDtypeStruct((NW, L), jnp.float32),
    ],
    scratch_types=[
        pltpu.VMEM((B,), jnp.float32),
        pltpu.VMEM((B,), jnp.int32),
        pltpu.VMEM((L,), jnp.float32),
        pltpu.VMEM((L,), jnp.float32),
    ],
)
def _rank_loss_partials(pred_hbm, rank_hbm, sum_hbm, cnt_hbm,
                        pred_v, rank_v, sacc_v, cacc_v):
    wid = lax.axis_index("s") * NC + lax.axis_index("c")
    pltpu.sync_copy(pred_hbm, pred_v)
    pltpu.sync_copy(rank_hbm, rank_v)
    base = wid * ROWS
    K = 4
    zero = jnp.zeros((L,), jnp.float32)

    def rowchunk_body(jc, carry):
        acc, cnt = carry
        pjv = pred_v[pl.ds(base + jc * L, L)]
        rjv = rank_v[pl.ds(base + jc * L, L)]
        for g in range(L // K):
            pjs = [jnp.full((L,), pjv[g * K + t]) for t in range(K)]
            rjs = [jnp.full((L,), rjv[g * K + t]) for t in range(K)]

            def chunk_body(c, carry2):
                accs, cnts = carry2
                pv = pred_v[pl.ds(c * L, L)]
                rv = rank_v[pl.ds(c * L, L)]
                t1 = 1.0 + pv
                accs = list(accs)
                cnts = list(cnts)
                for t in range(K):
                    m = rv < rjs[t]
                    d = jnp.maximum(t1 - pjs[t], 0.0)
                    accs[t] = accs[t] + jnp.where(m, d * d, 0.0)
                    cnts[t] = cnts[t] + jnp.where(m, 1.0, 0.0)
                return tuple(accs), tuple(cnts)

            accs, cnts = lax.fori_loop(
                0, CHUNKS, chunk_body,
                ((zero,) * K, (zero,) * K))
            for t in range(K):
                acc = acc + accs[t]
                cnt = cnt + cnts[t]
        return acc, cnt

    acc, cnt = lax.fori_loop(0, ROWS // L, rowchunk_body, (zero, zero))
    sacc_v[...] = acc
    cacc_v[...] = cnt
    pltpu.sync_copy(sacc_v, sum_hbm.at[wid])
    pltpu.sync_copy(cacc_v, cnt_hbm.at[wid])


def kernel(pred, rank_batch):
    sums, cnts = _rank_loss_partials(pred, rank_batch.astype(jnp.int32))
    return jnp.sum(sums) / jnp.sum(cnts)
